# baseline (speedup 1.0000x reference)
import functools

import jax
import jax.numpy as jnp
from jax import lax
from jax.experimental import pallas as pl
from jax.experimental.pallas import tpu as pltpu
from jax.experimental.pallas import tpu_sc as plsc

N = 10000
E = 320000
D = 128
DE = 16

NC = 2
NS = 16
NW = NC * NS
EPW = E // NW

CH = 16
NBUF = 5
NCHUNK = EPW // CH
NGRP = NCHUNK // NBUF
NSL = D // 16

NRC = 400
NRJ = N // NRC

_mesh = plsc.VectorSubcoreMesh(core_axis_name="c", subcore_axis_name="s")


@functools.partial(
    pl.kernel,
    out_type=[
        jax.ShapeDtypeStruct((NC, N, D), jnp.float32),
        jax.ShapeDtypeStruct((NC, N), jnp.float32),
    ],
    mesh=_mesh,
    scratch_types=[
        pltpu.VMEM((NBUF, CH), jnp.int32),
        pltpu.VMEM((NBUF, CH), jnp.int32),
        pltpu.VMEM((NBUF, CH, D), jnp.float32),
        pltpu.VMEM((NBUF, CH, D), jnp.float32),
        pltpu.VMEM((NBUF, CH, D), jnp.float32),
        pltpu.VMEM((NBUF, CH), jnp.float32),
        pltpu.VMEM((D,), jnp.float32),
        pltpu.VMEM_SHARED((N, D), jnp.float32),
        pltpu.VMEM_SHARED((N,), jnp.float32),
        pltpu.SemaphoreType.DMA((NBUF,)),
        pltpu.SemaphoreType.DMA((NBUF,)),
        pltpu.SemaphoreType.DMA((NBUF,)),
    ],
)
def _sc_fused(xl_hbm, xr_hbm, ep_hbm, src_hbm, dst_hbm, zacc_hbm, zden_hbm,
              att_hbm, acc_out, den_out,
              srcb, dstb, xlb, xrb, eb, pb, attb, acc_sp, den_sp,
              sem_i, sem_g, sem_s):
    c = lax.axis_index("c")
    s = lax.axis_index("s")
    wid = s * NC + c
    w0 = wid * EPW

    def initj(j, carry):
        @pl.when(j % NS == s)
        def _():
            pltpu.sync_copy(zacc_hbm.at[pl.ds(j * NRC, NRC)],
                            acc_sp.at[pl.ds(j * NRC, NRC)])
        return carry

    lax.fori_loop(0, NRJ, initj, 0)

    @pl.when(s == 0)
    def _initd():
        pltpu.sync_copy(zden_hbm, den_sp)

    pltpu.sync_copy(att_hbm, attb)
    plsc.subcore_barrier()

    def fire_idx(b, i):
        pltpu.async_copy(src_hbm.at[pl.ds(w0 + i * CH, CH)], srcb.at[b],
                         sem_i.at[b])
        pltpu.async_copy(dst_hbm.at[pl.ds(w0 + i * CH, CH)], dstb.at[b],
                         sem_i.at[b])

    for b in range(NBUF):
        fire_idx(b, b)

    def group(g, carry):
        for b in range(NBUF):
            i = g * NBUF + b
            pltpu.make_async_copy(
                src_hbm.at[pl.ds(0, CH)], srcb.at[b], sem_i.at[b]).wait()
            pltpu.make_async_copy(
                dst_hbm.at[pl.ds(0, CH)], dstb.at[b], sem_i.at[b]).wait()
            pltpu.async_copy(xl_hbm.at[srcb.at[b]], xlb.at[b], sem_g.at[b])
            pltpu.async_copy(xr_hbm.at[dstb.at[b]], xrb.at[b], sem_g.at[b])
            pltpu.async_copy(ep_hbm.at[pl.ds(w0 + i * CH, CH)], eb.at[b],
                             sem_g.at[b])
        for b in range(NBUF):
            pltpu.make_async_copy(
                xl_hbm.at[pl.ds(0, CH)], xlb.at[b], sem_g.at[b]).wait()
            pltpu.make_async_copy(
                xr_hbm.at[pl.ds(0, CH)], xrb.at[b], sem_g.at[b]).wait()
            pltpu.make_async_copy(
                ep_hbm.at[pl.ds(0, CH)], eb.at[b], sem_g.at[b]).wait()

            def edge(e, pgrp, b=b):
                acc = jnp.zeros((16,), jnp.float32)
                xls = []
                for k in range(NSL):
                    sl = pl.ds(k * 16, 16)
                    xlk = xlb[b, e, sl]
                    xls.append(xlk)
                    t = xlk + xrb[b, e, sl] + eb[b, e, sl]
                    t = jnp.maximum(t, 0.2 * t)
                    acc = acc + t * attb[sl]
                dn = lax.GatherDimensionNumbers(
                    offset_dims=(), collapsed_slice_dims=(0,),
                    start_index_map=(0,))
                for r in (1, 2, 4, 8):
                    perm = jnp.bitwise_xor(lax.iota(jnp.int32, 16), r)
                    acc = acc + lax.gather(
                        acc, perm[:, None], dn, slice_sizes=(1,),
                        mode=lax.GatherScatterMode.PROMISE_IN_BOUNDS)
                pv = jnp.exp(acc)
                for k in range(NSL):
                    sl = pl.ds(k * 16, 16)
                    xrb[b, e, sl] = xls[k] * pv
                return jnp.where(lax.iota(jnp.int32, 16) == e, pv, pgrp)

            pgrp = lax.fori_loop(0, CH, edge, jnp.zeros((16,), jnp.float32))
            pb[b, pl.ds(0, CH)] = pgrp
            pltpu.async_copy(xrb.at[b], acc_sp.at[dstb.at[b]], sem_s.at[b],
                             add=True)
            pltpu.async_copy(pb.at[b], den_sp.at[dstb.at[b]], sem_s.at[b],
                             add=True)
        for b in range(NBUF):
            i = g * NBUF + b
            pltpu.make_async_copy(
                xrb.at[b], acc_sp.at[pl.ds(0, CH)], sem_s.at[b]).wait()
            pltpu.make_async_copy(
                pb.at[b], den_sp.at[pl.ds(0, CH)], sem_s.at[b]).wait()

            @pl.when(g < NGRP - 1)
            def _prefetch(b=b, i=i):
                fire_idx(b, i + NBUF)

        return carry

    lax.fori_loop(0, NGRP, group, 0)

    plsc.subcore_barrier()

    def outj(j, carry):
        @pl.when(j % NS == s)
        def _():
            pltpu.sync_copy(acc_sp.at[pl.ds(j * NRC, NRC)],
                            acc_out.at[c, pl.ds(j * NRC, NRC)])
        return carry

    lax.fori_loop(0, NRJ, outj, 0)

    @pl.when(s == 0)
    def _outd():
        pltpu.sync_copy(den_sp, den_out.at[c])


def _proj_body(x_ref, wl_ref, wr_ref, xl_ref, xr_ref):
    x = x_ref[...]
    xl_ref[...] = jnp.dot(x, wl_ref[...], preferred_element_type=jnp.float32)
    xr_ref[...] = jnp.dot(x, wr_ref[...], preferred_element_type=jnp.float32)


def _tc_proj(h, wl, wr):
    return pl.pallas_call(
        _proj_body,
        out_shape=[
            jax.ShapeDtypeStruct((N, D), jnp.float32),
            jax.ShapeDtypeStruct((N, D), jnp.float32),
        ],
    )(h, wl, wr)


EB = 2560


def _eproj_body(ea_ref, we_ref, ep_ref):
    ep_ref[...] = jnp.dot(ea_ref[...], we_ref[...],
                          preferred_element_type=jnp.float32)


def _tc_eproj(ea, we):
    return pl.pallas_call(
        _eproj_body,
        grid=(E // EB,),
        in_specs=[
            pl.BlockSpec((EB, DE), lambda i: (i, 0)),
            pl.BlockSpec((DE, D), lambda i: (0, 0)),
        ],
        out_specs=pl.BlockSpec((EB, D), lambda i: (i, 0)),
        out_shape=jax.ShapeDtypeStruct((E, D), jnp.float32),
    )(ea, we)


def _norm_core(acc_ref, den_ref, b_ref, g_ref, be_ref):
    acc_t = acc_ref[0] + acc_ref[1]
    den_col = lax.dot_general(den_ref[...], jnp.ones((NC, 1), jnp.float32),
                              (((0,), (0,)), ((), ())),
                              preferred_element_type=jnp.float32)
    h = acc_t / (den_col + 1e-16) + b_ref[...]
    mu = jnp.mean(h, axis=0, keepdims=True)
    var = jnp.mean((h - mu) ** 2, axis=0, keepdims=True)
    hn = g_ref[...] * (h - mu) * lax.rsqrt(var + 1e-5) + be_ref[...]
    return jnp.maximum(hn, 0.0)


def _norm_proj_body(acc_ref, den_ref, b_ref, g_ref, be_ref, wl_ref, wr_ref,
                    xl_ref, xr_ref):
    hn = _norm_core(acc_ref, den_ref, b_ref, g_ref, be_ref)
    xl_ref[...] = jnp.dot(hn, wl_ref[...], preferred_element_type=jnp.float32)
    xr_ref[...] = jnp.dot(hn, wr_ref[...], preferred_element_type=jnp.float32)


def _tc_norm_proj(acc, den, b, g, be, wl, wr):
    return pl.pallas_call(
        _norm_proj_body,
        out_shape=[
            jax.ShapeDtypeStruct((N, D), jnp.float32),
            jax.ShapeDtypeStruct((N, D), jnp.float32),
        ],
    )(acc, den, b.reshape(1, D), g.reshape(1, D), be.reshape(1, D), wl, wr)


def _norm_fc_body(acc_ref, den_ref, b_ref, g_ref, be_ref, wfc_ref, bfc_ref,
                  o_ref):
    hn = _norm_core(acc_ref, den_ref, b_ref, g_ref, be_ref)
    o_ref[...] = jnp.dot(hn, wfc_ref[...],
                         preferred_element_type=jnp.float32) + bfc_ref[...]


def _tc_norm_fc(acc, den, b, g, be, wfc, bfc):
    return pl.pallas_call(
        _norm_fc_body,
        out_shape=jax.ShapeDtypeStruct((N, D), jnp.float32),
    )(acc, den, b.reshape(1, D), g.reshape(1, D), be.reshape(1, D), wfc,
      bfc.reshape(1, D))


def kernel(x, edge_index, edge_attr, batch, Wl0, Wr0, We0, att0, b0, g0, be0,
           Wl1, Wr1, We1, att1, b1, g1, be1, Wfc, bfc):
    src = edge_index[0]
    dst = edge_index[1]
    zacc = jnp.zeros((N, D), jnp.float32)
    zden = jnp.zeros((N,), jnp.float32)

    ep0 = _tc_eproj(edge_attr, We0)
    ep1 = _tc_eproj(edge_attr, We1)

    xl0, xr0 = _tc_proj(x, Wl0, Wr0)
    acc0, den0 = _sc_fused(xl0, xr0, ep0, src, dst, zacc, zden, att0)
    xl1, xr1 = _tc_norm_proj(acc0, den0, b0, g0, be0, Wl1, Wr1)
    acc1, den1 = _sc_fused(xl1, xr1, ep1, src, dst, zacc, zden, att1)
    return _tc_norm_fc(acc1, den1, b1, g1, be1, Wfc, bfc)


# speedup vs baseline: 1.0007x; 1.0007x over previous
import functools

import jax
import jax.numpy as jnp
from jax import lax
from jax.experimental import pallas as pl
from jax.experimental.pallas import tpu as pltpu
from jax.experimental.pallas import tpu_sc as plsc

N = 10000
E = 320000
D = 128
DE = 16

NC = 2
NS = 16
NW = NC * NS
EPW = E // NW

CH = 16
NBUF = 5
NCHUNK = EPW // CH
NGRP = NCHUNK // NBUF
NSL = D // 16

NRC = 400
NRJ = N // NRC

_mesh = plsc.VectorSubcoreMesh(core_axis_name="c", subcore_axis_name="s")


@functools.partial(
    pl.kernel,
    out_type=[
        jax.ShapeDtypeStruct((NC, N, D), jnp.float32),
        jax.ShapeDtypeStruct((NC, N), jnp.float32),
    ],
    mesh=_mesh,
    scratch_types=[
        pltpu.VMEM((NBUF, CH), jnp.int32),
        pltpu.VMEM((NBUF, CH), jnp.int32),
        pltpu.VMEM((NBUF, CH, D), jnp.float32),
        pltpu.VMEM((NBUF, CH, D), jnp.float32),
        pltpu.VMEM((NBUF, CH, D), jnp.float32),
        pltpu.VMEM((NBUF, CH), jnp.float32),
        pltpu.VMEM((D,), jnp.float32),
        pltpu.VMEM_SHARED((N, D), jnp.float32),
        pltpu.VMEM_SHARED((N,), jnp.float32),
        pltpu.SemaphoreType.DMA((NBUF,)),
        pltpu.SemaphoreType.DMA((NBUF,)),
        pltpu.SemaphoreType.DMA((NBUF,)),
    ],
)
def _sc_fused(xl_hbm, xr_hbm, ep_hbm, src_hbm, dst_hbm, zacc_hbm, zden_hbm,
              att_hbm, acc_out, den_out,
              srcb, dstb, xlb, xrb, eb, pb, attb, acc_sp, den_sp,
              sem_i, sem_g, sem_s):
    c = lax.axis_index("c")
    s = lax.axis_index("s")
    wid = s * NC + c
    w0 = wid * EPW

    def initj(j, carry):
        @pl.when(j % NS == s)
        def _():
            pltpu.sync_copy(zacc_hbm.at[pl.ds(j * NRC, NRC)],
                            acc_sp.at[pl.ds(j * NRC, NRC)])
        return carry

    lax.fori_loop(0, NRJ, initj, 0)

    @pl.when(s == 0)
    def _initd():
        pltpu.sync_copy(zden_hbm, den_sp)

    pltpu.sync_copy(att_hbm, attb)
    plsc.subcore_barrier()

    def fire_idx(b, i):
        pltpu.async_copy(src_hbm.at[pl.ds(w0 + i * CH, CH)], srcb.at[b],
                         sem_i.at[b])
        pltpu.async_copy(dst_hbm.at[pl.ds(w0 + i * CH, CH)], dstb.at[b],
                         sem_i.at[b])

    for b in range(NBUF):
        fire_idx(b, b)

    def group(g, carry):
        for b in range(NBUF):
            i = g * NBUF + b
            pltpu.make_async_copy(
                src_hbm.at[pl.ds(0, CH)], srcb.at[b], sem_i.at[b]).wait()
            pltpu.make_async_copy(
                dst_hbm.at[pl.ds(0, CH)], dstb.at[b], sem_i.at[b]).wait()
            pltpu.async_copy(xl_hbm.at[srcb.at[b]], xlb.at[b], sem_g.at[b])
            pltpu.async_copy(xr_hbm.at[dstb.at[b]], xrb.at[b], sem_g.at[b])
            pltpu.async_copy(ep_hbm.at[pl.ds(w0 + i * CH, CH)], eb.at[b],
                             sem_g.at[b])
        for b in range(NBUF):
            pltpu.make_async_copy(
                xl_hbm.at[pl.ds(0, CH)], xlb.at[b], sem_g.at[b]).wait()
            pltpu.make_async_copy(
                xr_hbm.at[pl.ds(0, CH)], xrb.at[b], sem_g.at[b]).wait()
            pltpu.make_async_copy(
                ep_hbm.at[pl.ds(0, CH)], eb.at[b], sem_g.at[b]).wait()

            @plsc.parallel_loop(0, CH, 1, unroll=4,
                                carry=jnp.zeros((16,), jnp.float32))
            def pgrp(e, pgrp, b=b):
                acc = jnp.zeros((16,), jnp.float32)
                xls = []
                for k in range(NSL):
                    sl = pl.ds(k * 16, 16)
                    xlk = xlb[b, e, sl]
                    xls.append(xlk)
                    t = xlk + xrb[b, e, sl] + eb[b, e, sl]
                    t = jnp.maximum(t, 0.2 * t)
                    acc = acc + t * attb[sl]
                dn = lax.GatherDimensionNumbers(
                    offset_dims=(), collapsed_slice_dims=(0,),
                    start_index_map=(0,))
                for r in (1, 2, 4, 8):
                    perm = jnp.bitwise_xor(lax.iota(jnp.int32, 16), r)
                    acc = acc + lax.gather(
                        acc, perm[:, None], dn, slice_sizes=(1,),
                        mode=lax.GatherScatterMode.PROMISE_IN_BOUNDS)
                pv = jnp.exp(acc)
                for k in range(NSL):
                    sl = pl.ds(k * 16, 16)
                    xrb[b, e, sl] = xls[k] * pv
                return jnp.where(lax.iota(jnp.int32, 16) == e, pv, pgrp)

            pb[b, pl.ds(0, CH)] = pgrp
            pltpu.async_copy(xrb.at[b], acc_sp.at[dstb.at[b]], sem_s.at[b],
                             add=True)
            pltpu.async_copy(pb.at[b], den_sp.at[dstb.at[b]], sem_s.at[b],
                             add=True)
        for b in range(NBUF):
            i = g * NBUF + b
            pltpu.make_async_copy(
                xrb.at[b], acc_sp.at[pl.ds(0, CH)], sem_s.at[b]).wait()
            pltpu.make_async_copy(
                pb.at[b], den_sp.at[pl.ds(0, CH)], sem_s.at[b]).wait()

            @pl.when(g < NGRP - 1)
            def _prefetch(b=b, i=i):
                fire_idx(b, i + NBUF)

        return carry

    lax.fori_loop(0, NGRP, group, 0)

    plsc.subcore_barrier()

    def outj(j, carry):
        @pl.when(j % NS == s)
        def _():
            pltpu.sync_copy(acc_sp.at[pl.ds(j * NRC, NRC)],
                            acc_out.at[c, pl.ds(j * NRC, NRC)])
        return carry

    lax.fori_loop(0, NRJ, outj, 0)

    @pl.when(s == 0)
    def _outd():
        pltpu.sync_copy(den_sp, den_out.at[c])


def _proj_body(x_ref, wl_ref, wr_ref, xl_ref, xr_ref):
    x = x_ref[...]
    xl_ref[...] = jnp.dot(x, wl_ref[...], preferred_element_type=jnp.float32)
    xr_ref[...] = jnp.dot(x, wr_ref[...], preferred_element_type=jnp.float32)


def _tc_proj(h, wl, wr):
    return pl.pallas_call(
        _proj_body,
        out_shape=[
            jax.ShapeDtypeStruct((N, D), jnp.float32),
            jax.ShapeDtypeStruct((N, D), jnp.float32),
        ],
    )(h, wl, wr)


EB = 2560


def _eproj_body(ea_ref, we_ref, ep_ref):
    ep_ref[...] = jnp.dot(ea_ref[...], we_ref[...],
                          preferred_element_type=jnp.float32)


def _tc_eproj(ea, we):
    return pl.pallas_call(
        _eproj_body,
        grid=(E // EB,),
        in_specs=[
            pl.BlockSpec((EB, DE), lambda i: (i, 0)),
            pl.BlockSpec((DE, D), lambda i: (0, 0)),
        ],
        out_specs=pl.BlockSpec((EB, D), lambda i: (i, 0)),
        out_shape=jax.ShapeDtypeStruct((E, D), jnp.float32),
    )(ea, we)


def _norm_core(acc_ref, den_ref, b_ref, g_ref, be_ref):
    acc_t = acc_ref[0] + acc_ref[1]
    den_col = lax.dot_general(den_ref[...], jnp.ones((NC, 1), jnp.float32),
                              (((0,), (0,)), ((), ())),
                              preferred_element_type=jnp.float32)
    h = acc_t / (den_col + 1e-16) + b_ref[...]
    mu = jnp.mean(h, axis=0, keepdims=True)
    var = jnp.mean((h - mu) ** 2, axis=0, keepdims=True)
    hn = g_ref[...] * (h - mu) * lax.rsqrt(var + 1e-5) + be_ref[...]
    return jnp.maximum(hn, 0.0)


def _norm_proj_body(acc_ref, den_ref, b_ref, g_ref, be_ref, wl_ref, wr_ref,
                    xl_ref, xr_ref):
    hn = _norm_core(acc_ref, den_ref, b_ref, g_ref, be_ref)
    xl_ref[...] = jnp.dot(hn, wl_ref[...], preferred_element_type=jnp.float32)
    xr_ref[...] = jnp.dot(hn, wr_ref[...], preferred_element_type=jnp.float32)


def _tc_norm_proj(acc, den, b, g, be, wl, wr):
    return pl.pallas_call(
        _norm_proj_body,
        out_shape=[
            jax.ShapeDtypeStruct((N, D), jnp.float32),
            jax.ShapeDtypeStruct((N, D), jnp.float32),
        ],
    )(acc, den, b.reshape(1, D), g.reshape(1, D), be.reshape(1, D), wl, wr)


def _norm_fc_body(acc_ref, den_ref, b_ref, g_ref, be_ref, wfc_ref, bfc_ref,
                  o_ref):
    hn = _norm_core(acc_ref, den_ref, b_ref, g_ref, be_ref)
    o_ref[...] = jnp.dot(hn, wfc_ref[...],
                         preferred_element_type=jnp.float32) + bfc_ref[...]


def _tc_norm_fc(acc, den, b, g, be, wfc, bfc):
    return pl.pallas_call(
        _norm_fc_body,
        out_shape=jax.ShapeDtypeStruct((N, D), jnp.float32),
    )(acc, den, b.reshape(1, D), g.reshape(1, D), be.reshape(1, D), wfc,
      bfc.reshape(1, D))


def kernel(x, edge_index, edge_attr, batch, Wl0, Wr0, We0, att0, b0, g0, be0,
           Wl1, Wr1, We1, att1, b1, g1, be1, Wfc, bfc):
    src = edge_index[0]
    dst = edge_index[1]
    zacc = jnp.zeros((N, D), jnp.float32)
    zden = jnp.zeros((N,), jnp.float32)

    ep0 = _tc_eproj(edge_attr, We0)
    ep1 = _tc_eproj(edge_attr, We1)

    xl0, xr0 = _tc_proj(x, Wl0, Wr0)
    acc0, den0 = _sc_fused(xl0, xr0, ep0, src, dst, zacc, zden, att0)
    xl1, xr1 = _tc_norm_proj(acc0, den0, b0, g0, be0, Wl1, Wr1)
    acc1, den1 = _sc_fused(xl1, xr1, ep1, src, dst, zacc, zden, att1)
    return _tc_norm_fc(acc1, den1, b1, g1, be1, Wfc, bfc)


# speedup vs baseline: 1.3333x; 1.3323x over previous
import functools

import jax
import jax.numpy as jnp
from jax import lax
from jax.experimental import pallas as pl
from jax.experimental.pallas import tpu as pltpu
from jax.experimental.pallas import tpu_sc as plsc

N = 10000
E = 320000
D = 128
DE = 16

NC = 2
NS = 16
NW = NC * NS
EPW = E // NW
CH = 80
NCHUNK = EPW // CH

_mesh = plsc.VectorSubcoreMesh(core_axis_name="c", subcore_axis_name="s")


NBUF = 5
NGRP = NCHUNK // NBUF


@functools.partial(
    pl.kernel,
    out_type=[
        jax.ShapeDtypeStruct((E, D), jnp.float32),
        jax.ShapeDtypeStruct((E, D), jnp.float32),
    ],
    mesh=_mesh,
    scratch_types=[
        pltpu.VMEM((NBUF, CH), jnp.int32),
        pltpu.VMEM((NBUF, CH), jnp.int32),
        pltpu.VMEM((NBUF, CH, D), jnp.float32),
        pltpu.VMEM((NBUF, CH, D), jnp.float32),
        pltpu.SemaphoreType.DMA((NBUF,)),
        pltpu.SemaphoreType.DMA((NBUF,)),
        pltpu.SemaphoreType.DMA((NBUF,)),
    ],
)
def _sc_gather(src_hbm, dst_hbm, xl_hbm, xr_hbm, xls_out, xrd_out,
               idx_s, idx_d, buf_a, buf_b, sem_i, sem_g, sem_w):
    c = lax.axis_index("c")
    s = lax.axis_index("s")
    wid = s * NC + c
    w0 = wid * EPW

    def fire_idx(b, i):
        base = w0 + i * CH
        pltpu.async_copy(src_hbm.at[pl.ds(base, CH)], idx_s.at[b], sem_i.at[b])
        pltpu.async_copy(dst_hbm.at[pl.ds(base, CH)], idx_d.at[b], sem_i.at[b])

    for b in range(NBUF):
        fire_idx(b, b)

    def group(g, carry):
        for b in range(NBUF):
            i = g * NBUF + b

            @pl.when(g > 0)
            def _wait_wb(b=b):
                pltpu.make_async_copy(
                    buf_a.at[b], xls_out.at[pl.ds(0, CH)], sem_w.at[b]).wait()
                pltpu.make_async_copy(
                    buf_b.at[b], xrd_out.at[pl.ds(0, CH)], sem_w.at[b]).wait()

            pltpu.make_async_copy(
                src_hbm.at[pl.ds(0, CH)], idx_s.at[b], sem_i.at[b]).wait()
            pltpu.make_async_copy(
                dst_hbm.at[pl.ds(0, CH)], idx_d.at[b], sem_i.at[b]).wait()
            pltpu.async_copy(xl_hbm.at[idx_s.at[b]], buf_a.at[b], sem_g.at[b])
            pltpu.async_copy(xr_hbm.at[idx_d.at[b]], buf_b.at[b], sem_g.at[b])
        for b in range(NBUF):
            i = g * NBUF + b
            base = w0 + i * CH
            pltpu.make_async_copy(
                xl_hbm.at[pl.ds(0, CH)], buf_a.at[b], sem_g.at[b]).wait()
            pltpu.make_async_copy(
                xr_hbm.at[pl.ds(0, CH)], buf_b.at[b], sem_g.at[b]).wait()
            pltpu.async_copy(buf_a.at[b], xls_out.at[pl.ds(base, CH)],
                             sem_w.at[b])
            pltpu.async_copy(buf_b.at[b], xrd_out.at[pl.ds(base, CH)],
                             sem_w.at[b])

            @pl.when(g < NGRP - 1)
            def _prefetch(b=b, i=i):
                fire_idx(b, i + NBUF)

        return carry

    lax.fori_loop(0, NGRP, group, 0)
    for b in range(NBUF):
        pltpu.make_async_copy(
            buf_a.at[b], xls_out.at[pl.ds(0, CH)], sem_w.at[b]).wait()
        pltpu.make_async_copy(
            buf_b.at[b], xrd_out.at[pl.ds(0, CH)], sem_w.at[b]).wait()


NRC = 400
NRJ = N // NRC

CHS = 40
NBUFS = 5
NCHUNKS = EPW // CHS
NGRPS = NCHUNKS // NBUFS


@functools.partial(
    pl.kernel,
    out_type=[
        jax.ShapeDtypeStruct((NC, N, D), jnp.float32),
        jax.ShapeDtypeStruct((NC, N), jnp.float32),
    ],
    mesh=_mesh,
    scratch_types=[
        pltpu.VMEM((NBUFS, CHS, D), jnp.float32),
        pltpu.VMEM((NBUFS, CHS), jnp.float32),
        pltpu.VMEM((NBUFS, CHS), jnp.int32),
        pltpu.VMEM_SHARED((N, D), jnp.float32),
        pltpu.VMEM_SHARED((N,), jnp.float32),
        pltpu.SemaphoreType.DMA((NBUFS,)),
        pltpu.SemaphoreType.DMA((NBUFS,)),
    ],
)
def _sc_scatter(s_hbm, p_hbm, dst_hbm, zacc_hbm, zden_hbm, acc_out, den_out,
                rows, pbuf, idx_d, acc_sp, den_sp, sem_l, sem_s):
    c = lax.axis_index("c")
    s = lax.axis_index("s")
    wid = s * NC + c
    w0 = wid * EPW

    def initj(j, carry):
        @pl.when(j % NS == s)
        def _():
            pltpu.sync_copy(zacc_hbm.at[pl.ds(j * NRC, NRC)],
                            acc_sp.at[pl.ds(j * NRC, NRC)])
        return carry

    lax.fori_loop(0, NRJ, initj, 0)

    @pl.when(s == 0)
    def _initd():
        pltpu.sync_copy(zden_hbm, den_sp)

    plsc.subcore_barrier()

    def fire_load(b, i):
        base = w0 + i * CHS
        pltpu.async_copy(dst_hbm.at[pl.ds(base, CHS)], idx_d.at[b],
                         sem_l.at[b])
        pltpu.async_copy(s_hbm.at[pl.ds(base, CHS)], rows.at[b], sem_l.at[b])
        pltpu.async_copy(p_hbm.at[pl.ds(base, CHS)], pbuf.at[b], sem_l.at[b])

    for b in range(NBUFS):
        fire_load(b, b)

    def group(g, carry):
        for b in range(NBUFS):
            pltpu.make_async_copy(
                dst_hbm.at[pl.ds(0, CHS)], idx_d.at[b], sem_l.at[b]).wait()
            pltpu.make_async_copy(
                s_hbm.at[pl.ds(0, CHS)], rows.at[b], sem_l.at[b]).wait()
            pltpu.make_async_copy(
                p_hbm.at[pl.ds(0, CHS)], pbuf.at[b], sem_l.at[b]).wait()
            pltpu.async_copy(rows.at[b], acc_sp.at[idx_d.at[b]], sem_s.at[b],
                             add=True)
            pltpu.async_copy(pbuf.at[b], den_sp.at[idx_d.at[b]], sem_s.at[b],
                             add=True)
        for b in range(NBUFS):
            i = g * NBUFS + b
            pltpu.make_async_copy(
                rows.at[b], acc_sp.at[pl.ds(0, CHS)], sem_s.at[b]).wait()
            pltpu.make_async_copy(
                pbuf.at[b], den_sp.at[pl.ds(0, CHS)], sem_s.at[b]).wait()

            @pl.when(g < NGRPS - 1)
            def _prefetch(b=b, i=i):
                fire_load(b, i + NBUFS)

        return carry

    lax.fori_loop(0, NGRPS, group, 0)

    plsc.subcore_barrier()

    def outj(j, carry):
        @pl.when(j % NS == s)
        def _():
            pltpu.sync_copy(acc_sp.at[pl.ds(j * NRC, NRC)],
                            acc_out.at[c, pl.ds(j * NRC, NRC)])
        return carry

    lax.fori_loop(0, NRJ, outj, 0)

    @pl.when(s == 0)
    def _outd():
        pltpu.sync_copy(den_sp, den_out.at[c])


def _proj_body(x_ref, wl_ref, wr_ref, xl_ref, xr_ref):
    x = x_ref[...]
    xl_ref[...] = jnp.dot(x, wl_ref[...], preferred_element_type=jnp.float32)
    xr_ref[...] = jnp.dot(x, wr_ref[...], preferred_element_type=jnp.float32)


def _tc_proj(h, wl, wr):
    return pl.pallas_call(
        _proj_body,
        out_shape=[
            jax.ShapeDtypeStruct((N, D), jnp.float32),
            jax.ShapeDtypeStruct((N, D), jnp.float32),
        ],
    )(h, wl, wr)


EB = 2560


def _alpha_body(xls_ref, xrd_ref, ea_ref, we_ref, attc_ref, attr_ref,
                s_ref, p_ref):
    xls = xls_ref[...]
    m = xls + xrd_ref[...] + jnp.dot(
        ea_ref[...], we_ref[...], preferred_element_type=jnp.float32)
    m = jnp.maximum(m, 0.2 * m)
    a_col = jnp.dot(m, attc_ref[...], preferred_element_type=jnp.float32)
    s_ref[...] = xls * jnp.exp(a_col)
    a_row = lax.dot_general(attr_ref[...], m, (((1,), (1,)), ((), ())),
                            preferred_element_type=jnp.float32)
    p_ref[...] = jnp.exp(a_row)[None]


def _tc_alpha(xls, xrd, ea, we, att):
    s_out, p3 = pl.pallas_call(
        _alpha_body,
        grid=(E // EB,),
        in_specs=[
            pl.BlockSpec((EB, D), lambda i: (i, 0)),
            pl.BlockSpec((EB, D), lambda i: (i, 0)),
            pl.BlockSpec((EB, DE), lambda i: (i, 0)),
            pl.BlockSpec((DE, D), lambda i: (0, 0)),
            pl.BlockSpec((D, 1), lambda i: (0, 0)),
            pl.BlockSpec((1, D), lambda i: (0, 0)),
        ],
        out_specs=[
            pl.BlockSpec((EB, D), lambda i: (i, 0)),
            pl.BlockSpec((1, 1, EB), lambda i: (i, 0, 0)),
        ],
        out_shape=[
            jax.ShapeDtypeStruct((E, D), jnp.float32),
            jax.ShapeDtypeStruct((E // EB, 1, EB), jnp.float32),
        ],
    )(xls, xrd, ea, we, att.reshape(D, 1), att.reshape(1, D))
    return s_out, p3.reshape(E)


def _norm_core(acc_ref, den_ref, b_ref, g_ref, be_ref):
    acc_t = acc_ref[0] + acc_ref[1]
    den_col = lax.dot_general(den_ref[...], jnp.ones((NC, 1), jnp.float32),
                              (((0,), (0,)), ((), ())),
                              preferred_element_type=jnp.float32)
    h = acc_t / (den_col + 1e-16) + b_ref[...]
    mu = jnp.mean(h, axis=0, keepdims=True)
    var = jnp.mean((h - mu) ** 2, axis=0, keepdims=True)
    hn = g_ref[...] * (h - mu) * lax.rsqrt(var + 1e-5) + be_ref[...]
    return jnp.maximum(hn, 0.0)


def _norm_proj_body(acc_ref, den_ref, b_ref, g_ref, be_ref, wl_ref, wr_ref,
                    xl_ref, xr_ref):
    hn = _norm_core(acc_ref, den_ref, b_ref, g_ref, be_ref)
    xl_ref[...] = jnp.dot(hn, wl_ref[...], preferred_element_type=jnp.float32)
    xr_ref[...] = jnp.dot(hn, wr_ref[...], preferred_element_type=jnp.float32)


def _tc_norm_proj(acc, den, b, g, be, wl, wr):
    return pl.pallas_call(
        _norm_proj_body,
        out_shape=[
            jax.ShapeDtypeStruct((N, D), jnp.float32),
            jax.ShapeDtypeStruct((N, D), jnp.float32),
        ],
    )(acc, den, b.reshape(1, D), g.reshape(1, D), be.reshape(1, D), wl, wr)


def _norm_fc_body(acc_ref, den_ref, b_ref, g_ref, be_ref, wfc_ref, bfc_ref,
                  o_ref):
    hn = _norm_core(acc_ref, den_ref, b_ref, g_ref, be_ref)
    o_ref[...] = jnp.dot(hn, wfc_ref[...],
                         preferred_element_type=jnp.float32) + bfc_ref[...]


def _tc_norm_fc(acc, den, b, g, be, wfc, bfc):
    return pl.pallas_call(
        _norm_fc_body,
        out_shape=jax.ShapeDtypeStruct((N, D), jnp.float32),
    )(acc, den, b.reshape(1, D), g.reshape(1, D), be.reshape(1, D), wfc,
      bfc.reshape(1, D))


def kernel(x, edge_index, edge_attr, batch, Wl0, Wr0, We0, att0, b0, g0, be0,
           Wl1, Wr1, We1, att1, b1, g1, be1, Wfc, bfc):
    src = edge_index[0]
    dst = edge_index[1]
    zacc = jnp.zeros((N, D), jnp.float32)
    zden = jnp.zeros((N,), jnp.float32)

    def layer(h, wl, wr, we, att):
        xl, xr = _tc_proj(h, wl, wr)
        xls, xrd = _sc_gather(src, dst, xl, xr)
        s_rows, p = _tc_alpha(xls, xrd, edge_attr, we, att)
        acc, den = _sc_scatter(s_rows, p, dst, zacc, zden)
        return acc, den

    acc0, den0 = layer(x, Wl0, Wr0, We0, att0)
    xl1_in = _tc_norm_proj(acc0, den0, b0, g0, be0, Wl1, Wr1)
    xls1, xrd1 = _sc_gather(src, dst, xl1_in[0], xl1_in[1])
    s1, p1 = _tc_alpha(xls1, xrd1, edge_attr, We1, att1)
    acc1, den1 = _sc_scatter(s1, p1, dst, zacc, zden)
    return _tc_norm_fc(acc1, den1, b1, g1, be1, Wfc, bfc)
